# Initial kernel scaffold; baseline (speedup 1.0000x reference)
import jax
import jax.numpy as jnp
import numpy as np
from jax import lax
from jax.experimental import pallas as pl
from jax.experimental.pallas import tpu as pltpu

D_MIN, D_MAX_RBF, N_BASES = 0.0, 4.5, 16
RADIUS = 5.0
G = 8
CHUNK = 128

_OFFSETS = np.linspace(D_MIN, D_MAX_RBF, N_BASES).astype(np.float32)
_COEFF = np.float32(-0.5 / (_OFFSETS[1] - _OFFSETS[0]) ** 2)


def _cfconv_body(lo_ref, nc_ref, feats_ref, csrc_ref, bsrc_ref, meta_ref,
                 w1_ref, w2_ref, out_ref):
    g = pl.program_id(0)
    lo = lo_ref[g]
    nc = nc_ref[g]

    meta = meta_ref[0]
    cd3 = meta[0:3, :]
    sqd = meta[3:4, :]
    bd = meta[4:5, :]
    gd = meta[5:6, :]

    offs = jnp.asarray(_OFFSETS).reshape(1, N_BASES)
    w1 = w1_ref[...]
    w2 = w2_ref[...]

    def chunk_body(c, acc):
        s0 = lo + c * CHUNK
        xs = feats_ref[pl.ds(s0, CHUNK), :]
        cs = csrc_ref[pl.ds(s0, CHUNK), :]
        bs = bsrc_ref[pl.ds(s0, CHUNK), :]
        cs3 = cs[:, 0:3]
        sqs = cs[:, 3:4]

        dots = jnp.dot(cs3, cd3, preferred_element_type=jnp.float32)
        d2 = jnp.maximum(sqs + sqd - 2.0 * dots, 0.0)
        d = jnp.sqrt(d2)

        sidx = jnp.float32(s0) + lax.broadcasted_iota(
            jnp.float32, (CHUNK, 1), 0)
        mask = ((bs == bd) & (sidx != gd)
                & (d2 <= RADIUS * RADIUS)).astype(jnp.float32)

        rbfs = jnp.concatenate(
            [jnp.exp(_COEFF * (d[:, j:j + 1] - offs) ** 2) for j in range(G)],
            axis=0)
        h = jax.nn.relu(jnp.dot(rbfs, w1, preferred_element_type=jnp.float32))
        m = jax.nn.relu(jnp.dot(h, w2, preferred_element_type=jnp.float32))

        rows = [
            jnp.sum(mask[:, j:j + 1] * xs * m[j * CHUNK:(j + 1) * CHUNK, :],
                    axis=0, keepdims=True)
            for j in range(G)
        ]
        return acc + jnp.concatenate(rows, axis=0)

    acc = jnp.zeros((G, 128), dtype=jnp.float32)
    out_ref[...] = lax.fori_loop(0, nc, chunk_body, acc)


@jax.jit
def kernel(node_feats, coords, batch_index, W1, W2):
    V, H = node_feats.shape
    b = batch_index.astype(jnp.int32)
    bf = b.astype(jnp.float32)

    VP = V + 2 * CHUNK
    pad = VP - V
    feats_p = jnp.pad(node_feats, ((0, pad), (0, 0)))
    sq = jnp.sum(coords * coords, axis=-1)
    csrc = jnp.concatenate(
        [coords, sq[:, None], jnp.zeros((V, 4), jnp.float32)], axis=1)
    csrc_p = jnp.pad(csrc, ((0, pad), (0, 0)))
    bsrc_p = jnp.pad(bf[:, None], ((0, pad), (0, 0)), constant_values=-7.0)

    num_groups = V // G
    gidx = jnp.arange(V, dtype=jnp.float32)
    zeros = jnp.zeros((V,), jnp.float32)
    meta = jnp.stack([coords[:, 0], coords[:, 1], coords[:, 2], sq,
                      bf, gidx, zeros, zeros], axis=0)
    meta = meta.reshape(8, num_groups, G).transpose(1, 0, 2)

    br = b.reshape(num_groups, G)
    lo = jnp.searchsorted(b, br[:, 0], side='left').astype(jnp.int32)
    hi = jnp.searchsorted(b, br[:, G - 1], side='right').astype(jnp.int32)
    lo8 = (lo // 8) * 8
    nchunks = ((hi - lo8 + CHUNK - 1) // CHUNK).astype(jnp.int32)

    grid_spec = pltpu.PrefetchScalarGridSpec(
        num_scalar_prefetch=2,
        grid=(num_groups,),
        in_specs=[
            pl.BlockSpec((VP, H), lambda g, *_: (0, 0)),
            pl.BlockSpec((VP, 8), lambda g, *_: (0, 0)),
            pl.BlockSpec((VP, 1), lambda g, *_: (0, 0)),
            pl.BlockSpec((1, 8, G), lambda g, *_: (g, 0, 0)),
            pl.BlockSpec((N_BASES, H), lambda g, *_: (0, 0)),
            pl.BlockSpec((H, H), lambda g, *_: (0, 0)),
        ],
        out_specs=pl.BlockSpec((G, H), lambda g, *_: (g, 0)),
    )

    out = pl.pallas_call(
        _cfconv_body,
        grid_spec=grid_spec,
        out_shape=jax.ShapeDtypeStruct((V, H), jnp.float32),
    )(lo8, nchunks, feats_p, csrc_p, bsrc_p, meta, W1, W2)
    return out


# speedup vs baseline: 130.4099x; 130.4099x over previous
import jax
import jax.numpy as jnp
import numpy as np
from jax import lax
from jax.experimental import pallas as pl
from jax.experimental.pallas import tpu as pltpu

D_MIN, D_MAX_RBF, N_BASES = 0.0, 4.5, 16
RADIUS = 5.0
G = 8
CHUNK = 128

_OFFSETS = np.linspace(D_MIN, D_MAX_RBF, N_BASES).astype(np.float32)
_COEFF = np.float32(-0.5 / (_OFFSETS[1] - _OFFSETS[0]) ** 2)


def _cfconv_body(lo_ref, nc_ref, feats_ref, csrc_ref, bsrc_ref, meta_ref,
                 w1_ref, w2_ref, out_ref):
    g = pl.program_id(0)
    lo = lo_ref[g]
    nc = nc_ref[g]

    meta = meta_ref[0]
    cdx = meta[0:1, :]
    cdy = meta[1:2, :]
    cdz = meta[2:3, :]
    bd = meta[3:4, :]
    gd = meta[4:5, :]

    step = np.float32((D_MAX_RBF - D_MIN) / (N_BASES - 1))
    offs = D_MIN + step * lax.broadcasted_iota(
        jnp.int32, (1, N_BASES), 1).astype(jnp.float32)
    w1 = w1_ref[...]
    w2 = w2_ref[...]

    def chunk_body(c, acc):
        s0 = lo + c * CHUNK
        xs = feats_ref[pl.ds(s0, CHUNK), :]
        cs = csrc_ref[pl.ds(s0, CHUNK), :]
        bs = bsrc_ref[pl.ds(s0, CHUNK), :]
        ddx = cs[:, 0:1] - cdx
        ddy = cs[:, 1:2] - cdy
        ddz = cs[:, 2:3] - cdz
        d2 = ddx * ddx + ddy * ddy + ddz * ddz
        d = jnp.sqrt(d2)

        sidx = s0.astype(jnp.float32) + lax.broadcasted_iota(
            jnp.int32, (CHUNK, 1), 0).astype(jnp.float32)
        mask = ((bs == bd) & (sidx != gd)
                & (d2 <= RADIUS * RADIUS)).astype(jnp.float32)

        rbfs = jnp.concatenate(
            [jnp.exp(_COEFF * (d[:, j:j + 1] - offs) ** 2) for j in range(G)],
            axis=0)
        h = jax.nn.relu(jnp.dot(rbfs, w1, preferred_element_type=jnp.float32))
        m = jax.nn.relu(jnp.dot(h, w2, preferred_element_type=jnp.float32))

        rows = [
            jnp.sum(mask[:, j:j + 1] * xs * m[j * CHUNK:(j + 1) * CHUNK, :],
                    axis=0, keepdims=True)
            for j in range(G)
        ]
        return acc + jnp.concatenate(rows, axis=0)

    acc = jnp.zeros((G, 128), dtype=jnp.float32)
    out_ref[...] = lax.fori_loop(0, nc, chunk_body, acc)


@jax.jit
def kernel(node_feats, coords, batch_index, W1, W2):
    V, H = node_feats.shape
    b = batch_index.astype(jnp.int32)
    bf = b.astype(jnp.float32)

    VP = V + 2 * CHUNK
    pad = VP - V
    feats_p = jnp.pad(node_feats, ((0, pad), (0, 0)))
    csrc = jnp.concatenate(
        [coords, jnp.zeros((V, 5), jnp.float32)], axis=1)
    csrc_p = jnp.pad(csrc, ((0, pad), (0, 0)))
    bsrc_p = jnp.pad(bf[:, None], ((0, pad), (0, 0)), constant_values=-7.0)

    num_groups = V // G
    gidx = jnp.arange(V, dtype=jnp.float32)
    zeros = jnp.zeros((V,), jnp.float32)
    meta = jnp.stack([coords[:, 0], coords[:, 1], coords[:, 2],
                      bf, gidx, zeros, zeros, zeros], axis=0)
    meta = meta.reshape(8, num_groups, G).transpose(1, 0, 2)

    br = b.reshape(num_groups, G)
    lo = jnp.searchsorted(b, br[:, 0], side='left').astype(jnp.int32)
    hi = jnp.searchsorted(b, br[:, G - 1], side='right').astype(jnp.int32)
    lo8 = (lo // 8) * 8
    nchunks = ((hi - lo8 + CHUNK - 1) // CHUNK).astype(jnp.int32)

    grid_spec = pltpu.PrefetchScalarGridSpec(
        num_scalar_prefetch=2,
        grid=(num_groups,),
        in_specs=[
            pl.BlockSpec((VP, H), lambda g, *_: (0, 0)),
            pl.BlockSpec((VP, 8), lambda g, *_: (0, 0)),
            pl.BlockSpec((VP, 1), lambda g, *_: (0, 0)),
            pl.BlockSpec((1, 8, G), lambda g, *_: (g, 0, 0)),
            pl.BlockSpec((N_BASES, H), lambda g, *_: (0, 0)),
            pl.BlockSpec((H, H), lambda g, *_: (0, 0)),
        ],
        out_specs=pl.BlockSpec((G, H), lambda g, *_: (g, 0)),
    )

    out = pl.pallas_call(
        _cfconv_body,
        grid_spec=grid_spec,
        out_shape=jax.ShapeDtypeStruct((V, H), jnp.float32),
    )(lo8, nchunks, feats_p, csrc_p, bsrc_p, meta, W1, W2)
    return out
